# baseline (speedup 1.0000x reference)
import jax
import jax.numpy as jnp
from jax.experimental import pallas as pl
from jax.experimental.pallas import tpu as pltpu

EPS = 1e-09

_ROWS = 100352
_COLS = 1024
_BLOCK_ROWS = 1024


def _scale_kernel(x_ref, o_ref, *, scale):
    o_ref[...] = x_ref[...] * scale


def kernel(x):
    H, W = x.shape[2], x.shape[3]
    scale = 1.0 / (H * W + EPS)
    x2 = x.reshape(_ROWS, _COLS)
    grid = (_ROWS // _BLOCK_ROWS,)
    out = pl.pallas_call(
        lambda x_ref, o_ref: _scale_kernel(x_ref, o_ref, scale=scale),
        grid=grid,
        in_specs=[pl.BlockSpec((_BLOCK_ROWS, _COLS), lambda i: (i, 0))],
        out_specs=pl.BlockSpec((_BLOCK_ROWS, _COLS), lambda i: (i, 0)),
        out_shape=jax.ShapeDtypeStruct((_ROWS, _COLS), x.dtype),
        compiler_params=pltpu.CompilerParams(
            dimension_semantics=("parallel",),
        ),
    )(x2)
    return out.reshape(x.shape)


# speedup vs baseline: 4.4746x; 4.4746x over previous
import jax
import jax.numpy as jnp
from jax.experimental import pallas as pl
from jax.experimental.pallas import tpu as pltpu

EPS = 1e-09

_BLOCK = 16


def _scale_kernel(x_ref, o_ref, *, scale):
    o_ref[...] = x_ref[...] * scale


def kernel(x):
    N, C, H, W = x.shape
    scale = 1.0 / (H * W + EPS)
    x3 = x.reshape(N * C, H, W)
    grid = ((N * C) // _BLOCK,)
    out = pl.pallas_call(
        lambda x_ref, o_ref: _scale_kernel(x_ref, o_ref, scale=scale),
        grid=grid,
        in_specs=[pl.BlockSpec((_BLOCK, H, W), lambda i: (i, 0, 0))],
        out_specs=pl.BlockSpec((_BLOCK, H, W), lambda i: (i, 0, 0)),
        out_shape=jax.ShapeDtypeStruct((N * C, H, W), x.dtype),
        compiler_params=pltpu.CompilerParams(
            dimension_semantics=("parallel",),
        ),
    )(x3)
    return out.reshape(x.shape)


# speedup vs baseline: 4.5307x; 1.0125x over previous
import jax
import jax.numpy as jnp
from jax.experimental import pallas as pl
from jax.experimental.pallas import tpu as pltpu

EPS = 1e-09

_BLOCK = 32


def _scale_kernel(x_ref, o_ref, *, scale):
    o_ref[...] = x_ref[...] * scale


def kernel(x):
    N, C, H, W = x.shape
    scale = 1.0 / (H * W + EPS)
    x3 = x.reshape(N * C, H, W)
    grid = ((N * C) // _BLOCK,)
    out = pl.pallas_call(
        lambda x_ref, o_ref: _scale_kernel(x_ref, o_ref, scale=scale),
        grid=grid,
        in_specs=[pl.BlockSpec((_BLOCK, H, W), lambda i: (i, 0, 0))],
        out_specs=pl.BlockSpec((_BLOCK, H, W), lambda i: (i, 0, 0)),
        out_shape=jax.ShapeDtypeStruct((N * C, H, W), x.dtype),
        compiler_params=pltpu.CompilerParams(
            dimension_semantics=("parallel",),
        ),
    )(x3)
    return out.reshape(x.shape)


# speedup vs baseline: 4.5605x; 1.0066x over previous
import jax
import jax.numpy as jnp
from jax.experimental import pallas as pl
from jax.experimental.pallas import tpu as pltpu

EPS = 1e-09

_BLOCK = 64


def _scale_kernel(x_ref, o_ref, *, scale):
    o_ref[...] = x_ref[...] * scale


def kernel(x):
    N, C, H, W = x.shape
    scale = 1.0 / (H * W + EPS)
    x3 = x.reshape(N * C, H, W)
    grid = ((N * C) // _BLOCK,)
    out = pl.pallas_call(
        lambda x_ref, o_ref: _scale_kernel(x_ref, o_ref, scale=scale),
        grid=grid,
        in_specs=[pl.BlockSpec((_BLOCK, H, W), lambda i: (i, 0, 0))],
        out_specs=pl.BlockSpec((_BLOCK, H, W), lambda i: (i, 0, 0)),
        out_shape=jax.ShapeDtypeStruct((N * C, H, W), x.dtype),
        compiler_params=pltpu.CompilerParams(
            dimension_semantics=("parallel",),
        ),
    )(x3)
    return out.reshape(x.shape)
